# baseline (speedup 1.0000x reference)
import functools

import jax
import jax.numpy as jnp
from jax import lax
from jax.experimental import pallas as pl
from jax.experimental.pallas import tpu as pltpu
from jax.experimental.pallas import tpu_sc as plsc

INPUT_DIM = 2048
HIDDEN_DIM = 16384
BATCH = 8192
K = 64

NWORKERS = 32
ROWS_PER_W = BATCH // NWORKERS
WORDS = 512
CAP = 128
RCH = 32
NCH = ROWS_PER_W // RCH


def _enc_body(x_ref, w_ref, b_ref, out_ref):
    acc = jax.lax.dot_general(
        x_ref[...], w_ref[...],
        dimension_numbers=(((1,), (1,)), ((), ())),
        preferred_element_type=jnp.float32)
    out_ref[...] = jnp.maximum(acc + b_ref[...], 0.0)


def _encode(xcb, Wb, b2):
    R = 512
    C = 2048
    grid = (HIDDEN_DIM // C, BATCH // R)
    return pl.pallas_call(
        _enc_body,
        grid=grid,
        in_specs=[
            pl.BlockSpec((R, INPUT_DIM), lambda c, r: (r, 0)),
            pl.BlockSpec((C, INPUT_DIM), lambda c, r: (c, 0)),
            pl.BlockSpec((1, C), lambda c, r: (0, c)),
        ],
        out_specs=pl.BlockSpec((R, C), lambda c, r: (r, c)),
        out_shape=jax.ShapeDtypeStruct((BATCH, HIDDEN_DIM), jnp.float32),
    )(xcb, Wb, b2)


def _screen_body(lat_ref, bm_ref):
    lat = lat_ref[...]
    R = lat.shape[0]
    NT = HIDDEN_DIM // WORDS
    li = jax.lax.bitcast_convert_type(lat, jnp.int32)
    m32 = li[:, :WORDS]
    for t in range(1, NT):
        m32 = jnp.maximum(m32, li[:, t * WORDS:(t + 1) * WORDS])

    def step(i, T):
        b = 30 - i
        cand = T | (1 << b)
        cnt = jnp.sum((m32 >= cand).astype(jnp.int32), axis=1, keepdims=True)
        return jnp.where(cnt >= K, cand, T)

    T = jax.lax.fori_loop(0, 31, step, jnp.zeros((R, 1), jnp.int32))
    bm = jnp.zeros((R, WORDS), jnp.int32)
    for t in range(NT):
        sl = li[:, t * WORDS:(t + 1) * WORDS]
        mask_t = (sl >= T) & (sl > 0)
        bm = bm | jnp.where(mask_t, jnp.int32(1) << t, 0)
    bm_ref[...] = bm


def _screen(latent):
    R = 256
    return pl.pallas_call(
        _screen_body,
        grid=(BATCH // R,),
        in_specs=[pl.BlockSpec((R, HIDDEN_DIM), lambda r: (r, 0))],
        out_specs=pl.BlockSpec((R, WORDS), lambda r: (r, 0)),
        out_shape=jax.ShapeDtypeStruct((BATCH, WORDS), jnp.int32),
    )(latent)


def _sc_fix_body(bm_hbm, lat_hbm, out_hbm,
                 bm_v, zrow_v, cidx_v, cval_v, hist_v, kidx_v, kval_v,
                 n_s, p_s, zsem, gsem, ssem):
    NC = 2
    wid = lax.axis_index("s") * NC + lax.axis_index("c")
    row0 = wid * ROWS_PER_W
    flat0 = row0 * HIDDEN_DIM
    iota16 = lax.iota(jnp.int32, 16)
    zeros16f = jnp.zeros((16,), jnp.float32)
    zeros16i = jnp.zeros((16,), jnp.int32)
    ones16i = jnp.ones((16,), jnp.int32)

    def _zinit(i, _):
        zrow_v[pl.ds(i * 16, 16)] = zeros16f
        return 0
    lax.fori_loop(0, HIDDEN_DIM // 16, _zinit, 0)

    def _cinit(i, _):
        r2 = i // (CAP // 16)
        c2 = i % (CAP // 16)
        cidx_v[r2, pl.ds(c2 * 16, 16)] = zeros16i
        return 0
    lax.fori_loop(0, RCH * CAP // 16, _cinit, 0)

    def _zfire(r, _):
        pltpu.async_copy(
            zrow_v, out_hbm.at[pl.ds(flat0 + r * HIDDEN_DIM, HIDDEN_DIM)],
            zsem)
        return 0
    lax.fori_loop(0, ROWS_PER_W, _zfire, 0)

    def chunk_loop(ch, _):
        rowc = row0 + ch * RCH
        pltpu.sync_copy(bm_hbm.at[pl.ds(rowc, RCH)], bm_v)

        def extract(rr, _):
            fb = (rowc + rr) * HIDDEN_DIM

            def vloop(i, carry):
                n, z = carry
                w0 = bm_v[rr, pl.ds(i * 16, 16)]
                jvec = i * 16 + iota16
                zm = w0 == 0
                zL = jnp.sum(jnp.where(iota16 == 0,
                                       plsc.all_reduce_ffs(zm), 0))
                z = jnp.where(jnp.any(zm) & (z < 0), i * 16 + zL, z)

                def cond(c):
                    return jnp.any(c[0] != 0)

                def body(c):
                    w2, n2 = c
                    m = w2 != 0
                    b = w2 & (-w2)
                    bf = jnp.abs(b.astype(jnp.float32))
                    t = (plsc.bitcast(bf, jnp.int32) >> 23) - 127
                    elem = t * WORDS + jvec
                    mi = m.astype(jnp.int32)
                    dst = n2 + plsc.cumsum(mi) - 1
                    mm = m & (dst < CAP)
                    rrv = jnp.broadcast_to(rr, (16,))
                    plsc.store_scatter(cidx_v, [rrv, dst], fb + elem, mask=mm)
                    return (w2 & (w2 - 1), n2 + jnp.sum(mi))

                w1, n = lax.while_loop(cond, body, (w0, n))
                return (n, z)

            n, z = lax.fori_loop(0, WORDS // 16, vloop,
                                 (jnp.int32(0), jnp.int32(-1)))
            n_s[rr] = jnp.minimum(n, CAP)
            p_s[rr] = fb + jnp.maximum(z, 0)
            pltpu.async_copy(lat_hbm.at[cidx_v.at[rr]], cval_v.at[rr], gsem)
            return 0

        lax.fori_loop(0, RCH, extract, 0)

        def gdrain(rr, _):
            pltpu.make_async_copy(
                lat_hbm.at[cidx_v.at[rr]], cval_v.at[rr], gsem).wait()
            return 0
        lax.fori_loop(0, RCH, gdrain, 0)

        def select(rr, _):
            n = n_s[rr]
            gslot = ch * RCH + rr
            gsv = jnp.broadcast_to(gslot, (16,))

            pref = jnp.int32(0)
            found0 = jnp.bool_(True)
            rem = jnp.int32(K)
            for lvl in range(4):
                sh = (23, 15, 7, 0)[lvl]
                shp = (31, 23, 15, 7)[lvl]
                bmask = 0x7F if lvl == 3 else 0xFF

                def _hzero(i, _):
                    hist_v[pl.ds(i * 16, 16)] = zeros16i
                    return 0
                lax.fori_loop(0, 16, _hzero, 0)

                def _hscan(i, _, sh=sh, shp=shp, bmask=bmask, lvl=lvl,
                           n=n, rr=rr, pref=pref):
                    v = cval_v[rr, pl.ds(i * 16, 16)]
                    vi = plsc.bitcast(v, jnp.int32)
                    m = (i * 16 + iota16) < n
                    if lvl > 0:
                        m = m & ((vi >> shp) == pref)
                    bucket = (vi >> sh) & bmask
                    plsc.addupdate_scatter(hist_v, [bucket], ones16i, mask=m)
                    return 0
                lax.fori_loop(0, CAP // 16, _hscan, 0)

                def _suf(i2, carry, rem=rem):
                    cacc, found, bstar, rnext = carry
                    i = 15 - i2
                    h = hist_v[pl.ds(i * 16, 16)]
                    hr = lax.rev(h, (0,))
                    c = plsc.cumsum(hr) + cacc
                    hitm = c >= rem
                    L = jnp.sum(jnp.where(iota16 == 0,
                                          plsc.all_reduce_ffs(hitm), 0))
                    fh = jnp.any(hitm)
                    cL = jnp.sum(jnp.where(iota16 == L, c, 0))
                    hL = jnp.sum(jnp.where(iota16 == L, hr, 0))
                    bthis = i * 16 + (15 - L)
                    newb = fh & jnp.logical_not(found)
                    bstar = jnp.where(newb, bthis, bstar)
                    rnext = jnp.where(newb, rem - (cL - hL), rnext)
                    return (cacc + jnp.sum(hr), found | fh, bstar, rnext)

                _, fnd, bstar, rem = lax.fori_loop(
                    0, 16, _suf,
                    (jnp.int32(0), jnp.bool_(False), jnp.int32(0),
                     jnp.int32(1)))
                if lvl == 0:
                    found0 = fnd
                pref = (pref << 8) | bstar if lvl < 3 else (pref << 7) | bstar

            T = jnp.where(found0, pref, 0)

            padi = jnp.broadcast_to(p_s[rr], (16,))

            def _kinit(i, _):
                kidx_v[gslot, pl.ds(i * 16, 16)] = padi
                kval_v[gslot, pl.ds(i * 16, 16)] = zeros16f
                return 0
            lax.fori_loop(0, K // 16, _kinit, 0)

            def _keep(i, nk):
                v = cval_v[rr, pl.ds(i * 16, 16)]
                ci = cidx_v[rr, pl.ds(i * 16, 16)]
                vi = plsc.bitcast(v, jnp.int32)
                m = ((i * 16 + iota16) < n) & (vi >= T)
                mi = m.astype(jnp.int32)
                dst = nk + plsc.cumsum(mi) - 1
                mm = m & (dst < K)
                plsc.store_scatter(kidx_v, [gsv, dst], ci, mask=mm)
                plsc.store_scatter(kval_v, [gsv, dst], v, mask=mm)
                return nk + jnp.sum(mi)
            lax.fori_loop(0, CAP // 16, _keep, jnp.int32(0))
            return 0

        lax.fori_loop(0, RCH, select, 0)
        return 0

    lax.fori_loop(0, NCH, chunk_loop, 0)

    def _zdrain(r, _):
        pltpu.make_async_copy(
            zrow_v, out_hbm.at[pl.ds(flat0 + r * HIDDEN_DIM, HIDDEN_DIM)],
            zsem).wait()
        return 0
    lax.fori_loop(0, ROWS_PER_W, _zdrain, 0)

    def _sfire(g, _):
        pltpu.async_copy(kval_v.at[g], out_hbm.at[kidx_v.at[g]], ssem)
        return 0
    lax.fori_loop(0, ROWS_PER_W, _sfire, 0)

    def _sdrain(g, _):
        pltpu.make_async_copy(
            kval_v.at[g], out_hbm.at[kidx_v.at[g]], ssem).wait()
        return 0
    lax.fori_loop(0, ROWS_PER_W, _sdrain, 0)


def _sc_fix(bitmap, latent_flat):
    mesh = plsc.VectorSubcoreMesh(core_axis_name="c", subcore_axis_name="s")
    fix = pl.kernel(
        _sc_fix_body,
        out_type=jax.ShapeDtypeStruct((BATCH * HIDDEN_DIM,), jnp.float32),
        mesh=mesh,
        compiler_params=pltpu.CompilerParams(needs_layout_passes=False),
        scratch_types=[
            pltpu.VMEM((RCH, WORDS), jnp.int32),
            pltpu.VMEM((HIDDEN_DIM,), jnp.float32),
            pltpu.VMEM((RCH, CAP), jnp.int32),
            pltpu.VMEM((RCH, CAP), jnp.float32),
            pltpu.VMEM((256,), jnp.int32),
            pltpu.VMEM((ROWS_PER_W, K), jnp.int32),
            pltpu.VMEM((ROWS_PER_W, K), jnp.float32),
            pltpu.SMEM((RCH,), jnp.int32),
            pltpu.SMEM((RCH,), jnp.int32),
            pltpu.SemaphoreType.DMA,
            pltpu.SemaphoreType.DMA,
            pltpu.SemaphoreType.DMA,
        ],
    )
    return fix(bitmap, latent_flat)


def _dec_body(lat_ref, w_ref, b_ref, out_ref):
    k = pl.program_id(1)

    @pl.when(k == 0)
    def _():
        out_ref[...] = jnp.broadcast_to(b_ref[...], out_ref.shape)

    lat_bf = lat_ref[...].astype(jnp.bfloat16)
    out_ref[...] += jax.lax.dot_general(
        lat_bf, w_ref[...],
        dimension_numbers=(((1,), (1,)), ((), ())),
        preferred_element_type=jnp.float32)


def _decode(latent_sparse, Wdb, bias2):
    G = 1024
    Kc = 2048
    grid = (BATCH // G, HIDDEN_DIM // Kc)
    return pl.pallas_call(
        _dec_body,
        grid=grid,
        in_specs=[
            pl.BlockSpec((G, Kc), lambda g, k: (g, k)),
            pl.BlockSpec((INPUT_DIM, Kc), lambda g, k: (0, k)),
            pl.BlockSpec((1, INPUT_DIM), lambda g, k: (0, 0)),
        ],
        out_specs=pl.BlockSpec((G, INPUT_DIM), lambda g, k: (g, 0)),
        out_shape=jax.ShapeDtypeStruct((BATCH, INPUT_DIM), jnp.float32),
    )(latent_sparse, Wdb, bias2)


def kernel(x, W_enc, b_enc, W_dec, bias):
    xcb = (x - bias).astype(jnp.bfloat16)
    Wb = W_enc.astype(jnp.bfloat16)
    latent = _encode(xcb, Wb, b_enc.reshape(1, -1))
    bitmap = _screen(latent)
    ls_flat = _sc_fix(bitmap, latent.reshape(-1))
    latent_sparse = ls_flat.reshape(BATCH, HIDDEN_DIM)
    recon = _decode(latent_sparse, W_dec.astype(jnp.bfloat16),
                    bias.reshape(1, -1))
    return (latent_sparse, recon)
